# Initial kernel scaffold; baseline (speedup 1.0000x reference)
import jax
import jax.numpy as jnp
from jax.experimental import pallas as pl


def kernel(x, edge_w, current_vector, src, dst, can_index, obs_index):
    raise NotImplementedError("write your pallas kernel here")


# speedup vs baseline: 43.0548x; 43.0548x over previous
import functools

import jax
import jax.numpy as jnp
from jax import lax
from jax.experimental import pallas as pl
from jax.experimental.pallas import tpu as pltpu
from jax.experimental.pallas import tpu_sc as plsc

_N = 50000
_E = 1600000
_VDD = 1.8
_ITERS = 300
_NW = 32
_R = 1568
_NPAD = _NW * _R
_ECAP = 51968
_DCAP = 510
_METAW = 528


def _preprocess(x, edge_w, current_vector, src, dst, can_index):
    src = src.astype(jnp.int32)
    dst = dst.astype(jnp.int32)
    ones = jnp.ones((_E,), jnp.int32)
    cnt = jax.ops.segment_sum(ones, src, num_segments=_N)
    degw = jax.ops.segment_sum(edge_w, src, num_segments=_N)
    g = jnp.zeros((_N,), x.dtype).at[can_index].set(x)
    d = degw + g
    j = -current_vector + _VDD * g
    inv_d = 1.0 / d
    jd = j * inv_d

    cnt_p = jnp.concatenate([cnt, jnp.zeros((_NPAD - _N,), jnp.int32)])
    order = jnp.argsort(-cnt_p)
    q = jnp.arange(_NPAD, dtype=jnp.int32)
    new_of_q = (q % _NW) * _R + q // _NW
    rank = jnp.zeros((_NPAD,), jnp.int32).at[order].set(new_of_q)
    iperm = jnp.zeros((_NPAD,), jnp.int32).at[new_of_q].set(order)

    cnt_new = cnt_p[iperm]
    jdp = jnp.concatenate([jd, jnp.zeros((_NPAD - _N,), jnp.float32)])[iperm]

    src2 = rank[src]
    dst2 = rank[dst]
    wp = edge_w * inv_d[src]

    eord = jnp.argsort(src2)
    s_src = src2[eord]
    row_start = jnp.concatenate(
        [jnp.zeros((1,), jnp.int32), jnp.cumsum(cnt_new, dtype=jnp.int32)])
    t_sorted = jnp.arange(_E, dtype=jnp.int32) - row_start[s_src]
    t = jnp.zeros((_E,), jnp.int32).at[eord].set(t_sorted)

    cw = cnt_new.reshape(_NW, _R)
    ks = jnp.arange(_DCAP, dtype=jnp.int32)
    nk = jnp.sum(cw[:, :, None] > ks[None, None, :], axis=1, dtype=jnp.int32)
    nk_pad = ((nk + 31) // 32) * 32
    colstart = jnp.concatenate(
        [jnp.zeros((_NW, 1), jnp.int32), jnp.cumsum(nk_pad, axis=1, dtype=jnp.int32)],
        axis=1)
    ncols = jnp.minimum(cw[:, 0], _DCAP)
    meta = jnp.zeros((_NW, _METAW), jnp.int32)
    meta = meta.at[:, 0].set(ncols)
    meta = meta.at[:, 1:1 + _DCAP].set(nk_pad // 32)

    wrk = src2 // _R
    i_loc = src2 % _R
    t_c = jnp.minimum(t, _DCAP - 1)
    pos = colstart[wrk, t_c] + i_loc
    valid = (t < _DCAP) & (pos < _ECAP)
    oob = jnp.int32(2**30)
    gslot = jnp.where(valid, wrk * _ECAP + pos, oob)
    wflat = jnp.zeros((_NW * _ECAP,), jnp.float32).at[gslot].set(wp, mode="drop")

    r32 = pos % 32
    slot16 = (pos - r32) + 2 * (r32 % 16) + (r32 // 16)
    gslot16 = jnp.where(valid, wrk * _ECAP + slot16, oob)
    word = lax.shift_right_logical(gslot16, 1)
    sh = (gslot16 & 1) * 16
    dwords = jnp.zeros((_NW * _ECAP // 2,), jnp.int32).at[word].add(
        lax.shift_left(dst2, sh), mode="drop")
    return (dwords.reshape(_NW, _ECAP // 2), wflat.reshape(_NW, _ECAP),
            jdp, meta, rank)


def _sc_step(dwords, wflat, jdp, meta, v):
    mesh = plsc.VectorSubcoreMesh(core_axis_name="c", subcore_axis_name="s")

    @functools.partial(
        pl.kernel,
        out_type=jax.ShapeDtypeStruct((_NPAD,), jnp.float32),
        mesh=mesh,
        scratch_types=[
            pltpu.VMEM((_NPAD,), jnp.float32),
            pltpu.VMEM((_ECAP,), jnp.float32),
            pltpu.VMEM((_ECAP // 2,), jnp.int32),
            pltpu.VMEM((_R,), jnp.float32),
            pltpu.VMEM((_METAW,), jnp.int32),
        ],
        compiler_params=pltpu.CompilerParams(needs_layout_passes=False),
    )
    def step(dw_h, w_h, jd_h, mt_h, v_h, out_h, v_l, w_l, d_l, acc, mt_l):
        wid = lax.axis_index("s") * 2 + lax.axis_index("c")
        pltpu.sync_copy(v_h, v_l)
        pltpu.sync_copy(w_h.at[wid], w_l)
        pltpu.sync_copy(dw_h.at[wid], d_l)
        pltpu.sync_copy(mt_h.at[wid], mt_l)
        pltpu.sync_copy(jd_h.at[pl.ds(wid * _R, _R)], acc)
        ncols = mt_l[pl.ds(0, 16)][0]

        def col_body(k, colstart):
            nch = mt_l[pl.ds(1 + k, 16)][0]

            def ch_body(m, carry):
                base = colstart + m * 32
                dw = d_l[pl.ds(lax.shift_right_logical(base, 1), 16)]
                ilo = jnp.bitwise_and(dw, 0xFFFF)
                ihi = lax.shift_right_logical(dw, 16)
                va = plsc.load_gather(v_l, [ilo])
                vb = plsc.load_gather(v_l, [ihi])
                wa = w_l[pl.ds(base, 16)]
                wb = w_l[pl.ds(base + 16, 16)]
                ar = m * 32
                plsc.addupdate(acc.at[pl.ds(ar, 16)], wa * va)
                plsc.addupdate(acc.at[pl.ds(ar + 16, 16)], wb * vb)
                return carry

            lax.fori_loop(0, nch, ch_body, 0)
            return colstart + nch * 32

        lax.fori_loop(0, ncols, col_body, jnp.int32(0))
        pltpu.sync_copy(acc, out_h.at[pl.ds(wid * _R, _R)])

    return step(dwords, wflat, jdp, meta, v)


def kernel(x, edge_w, current_vector, src, dst, can_index, obs_index):
    dwords, wflat, jdp, meta, rank = _preprocess(
        x, edge_w, current_vector, src, dst, can_index)

    def body(v, _):
        return _sc_step(dwords, wflat, jdp, meta, v), None

    v0 = jnp.zeros((_NPAD,), jnp.float32)
    vf, _ = lax.scan(body, v0, None, length=_ITERS)
    return vf[rank[obs_index.astype(jnp.int32)]]
